# baseline (speedup 1.0000x reference)
import functools

import jax
import jax.numpy as jnp
from jax import lax
from jax.experimental import pallas as pl
from jax.experimental.pallas import tpu as pltpu
from jax.experimental.pallas import tpu_sc as plsc

EMB_DIM = 64
PAD_D = 128
NUM_CORES = 2
NUM_SUBCORES = 16
NUM_WORKERS = NUM_CORES * NUM_SUBCORES
N_BUF = 8
K_AHEAD = 6
PAD_BLK = 16000


def _pad_tc(wt):
    V = wt.shape[1]

    def body(wt_ref, o_ref):
        rows = wt_ref[...].T
        o_ref[...] = jnp.concatenate([rows, rows], axis=1)

    return pl.pallas_call(
        body,
        grid=(pl.cdiv(V, PAD_BLK),),
        in_specs=[pl.BlockSpec((EMB_DIM, PAD_BLK), lambda i: (0, i))],
        out_specs=pl.BlockSpec((PAD_BLK, PAD_D), lambda i: (i, 0)),
        out_shape=jax.ShapeDtypeStruct((V, PAD_D), jnp.float32),
    )(wt)


def _gather(x2, w2, cols):
    n_rows = x2.shape[0]
    rows_per_w = n_rows // NUM_WORKERS
    nblk = rows_per_w // N_BUF
    mesh = plsc.VectorSubcoreMesh(core_axis_name="c", subcore_axis_name="s")

    @functools.partial(
        pl.kernel,
        mesh=mesh,
        out_type=jax.ShapeDtypeStruct((n_rows, cols, EMB_DIM), jnp.float32),
        compiler_params=pltpu.CompilerParams(use_tc_tiling_on_sc=False),
        scratch_types=(
            [pltpu.VMEM((rows_per_w, cols), jnp.int32)]
            + [pltpu.VMEM((cols, EMB_DIM), jnp.float32)] * N_BUF
            + [pltpu.SemaphoreType.DMA] * (2 * N_BUF)
        ),
    )
    def k(x_hbm, w_hbm, out_hbm, idx_v, *rest):
        bufs = rest[:N_BUF]
        gsems = rest[N_BUF : 2 * N_BUF]
        osems = rest[2 * N_BUF :]
        wid = lax.axis_index("s") * NUM_CORES + lax.axis_index("c")
        base = wid * rows_per_w
        pltpu.sync_copy(x_hbm.at[pl.ds(base, rows_per_w)], idx_v)

        def gather_start(c, b):
            pltpu.make_async_copy(
                w_hbm.at[idx_v.at[c]], bufs[b], gsems[b]
            ).start()

        def gather_wait(b):
            pltpu.make_async_copy(
                w_hbm.at[idx_v.at[0]], bufs[b], gsems[b]
            ).wait()

        def store_start(c, b):
            pltpu.make_async_copy(
                bufs[b], out_hbm.at[base + c], osems[b]
            ).start()

        def store_wait(b):
            pltpu.make_async_copy(
                bufs[b], out_hbm.at[base], osems[b]
            ).wait()

        def block(jj, first=False, last=False):
            for b in range(N_BUF):
                c = jj * N_BUF + b
                gather_wait(b)
                store_start(c, b)
                bk = (b + K_AHEAD) % N_BUF
                if last and b >= N_BUF - K_AHEAD:
                    continue
                if not (first and b < N_BUF - K_AHEAD):
                    store_wait(bk)
                gather_start(c + K_AHEAD, bk)

        for c in range(K_AHEAD):
            gather_start(c, c)
        block(0, first=True)
        lax.fori_loop(1, nblk - 1, lambda jj, cr: (block(jj), cr)[1], 0)
        block(nblk - 1, last=True)
        for b in range(N_BUF):
            store_wait(b)

    return k(x2, w2)


@functools.partial(jax.jit, static_argnums=(2,))
def _embed(x, weights, cols):
    w128 = _pad_tc(weights.T)
    w2 = w128.reshape(2 * weights.shape[0], EMB_DIM)
    x2 = x * 2
    half = cols // 2
    out_a = _gather(x2[:, :half], w2, half)
    out_b = _gather(x2[:, half:], w2, cols - half)
    return jnp.concatenate([out_a, out_b], axis=1)


def kernel(x, weights):
    n_rows, cols = x.shape
    assert n_rows % (NUM_WORKERS * N_BUF) == 0
    assert cols <= 128
    return _embed(x.astype(jnp.int32), weights, cols)


# speedup vs baseline: 1.0859x; 1.0859x over previous
import functools

import jax
import jax.numpy as jnp
from jax import lax
from jax.experimental import pallas as pl
from jax.experimental.pallas import tpu as pltpu
from jax.experimental.pallas import tpu_sc as plsc

EMB_DIM = 64
PAD_D = 128
NUM_CORES = 2
NUM_SUBCORES = 16
NUM_WORKERS = NUM_CORES * NUM_SUBCORES
N_BUF = 8
K_AHEAD = 6
PAD_BLK = 16000


def _pad_tc(wt):
    V = wt.shape[1]

    def body(wt_ref, o_ref):
        rows = wt_ref[...].T
        o_ref[...] = jnp.concatenate([rows, rows], axis=1)

    return pl.pallas_call(
        body,
        grid=(pl.cdiv(V, PAD_BLK),),
        in_specs=[pl.BlockSpec((EMB_DIM, PAD_BLK), lambda i: (0, i))],
        out_specs=pl.BlockSpec((PAD_BLK, PAD_D), lambda i: (i, 0)),
        out_shape=jax.ShapeDtypeStruct((V, PAD_D), jnp.float32),
    )(wt)


def _gather(x2, w2, cols):
    n_rows = x2.shape[0]
    rows_per_w = n_rows // NUM_WORKERS
    nblk = rows_per_w // N_BUF
    mesh = plsc.VectorSubcoreMesh(core_axis_name="c", subcore_axis_name="s")

    @functools.partial(
        pl.kernel,
        mesh=mesh,
        out_type=jax.ShapeDtypeStruct((n_rows, cols, EMB_DIM), jnp.float32),
        compiler_params=pltpu.CompilerParams(use_tc_tiling_on_sc=False),
        scratch_types=(
            [pltpu.VMEM((rows_per_w, cols), jnp.int32)]
            + [pltpu.VMEM((cols, EMB_DIM), jnp.float32)] * N_BUF
            + [pltpu.SemaphoreType.DMA] * (2 * N_BUF)
        ),
    )
    def k(x_hbm, w_hbm, out_hbm, idx_v, *rest):
        bufs = rest[:N_BUF]
        gsems = rest[N_BUF : 2 * N_BUF]
        osems = rest[2 * N_BUF :]
        wid = lax.axis_index("s") * NUM_CORES + lax.axis_index("c")
        base = wid * rows_per_w
        pltpu.sync_copy(x_hbm.at[pl.ds(base, rows_per_w)], idx_v)

        def gather_start(c, b):
            pltpu.make_async_copy(
                w_hbm.at[idx_v.at[c]], bufs[b], gsems[b]
            ).start()

        def gather_wait(b):
            pltpu.make_async_copy(
                w_hbm.at[idx_v.at[0]], bufs[b], gsems[b]
            ).wait()

        def store_start(c, b):
            pltpu.make_async_copy(
                bufs[b], out_hbm.at[base + c], osems[b]
            ).start()

        def store_wait(b):
            pltpu.make_async_copy(
                bufs[b], out_hbm.at[base], osems[b]
            ).wait()

        def block(jj, first=False, last=False):
            for b in range(N_BUF):
                c = jj * N_BUF + b
                gather_wait(b)
                store_start(c, b)
                bk = (b + K_AHEAD) % N_BUF
                if last and b >= N_BUF - K_AHEAD:
                    continue
                if not (first and b < N_BUF - K_AHEAD):
                    store_wait(bk)
                gather_start(c + K_AHEAD, bk)

        for c in range(K_AHEAD):
            gather_start(c, c)
        block(0, first=True)
        lax.fori_loop(1, nblk - 1, lambda jj, cr: (block(jj), cr)[1], 0)
        block(nblk - 1, last=True)
        for b in range(N_BUF):
            store_wait(b)

    return k(x2, w2)


@functools.partial(jax.jit, static_argnums=(2,))
def _embed(x, weights, cols):
    w128 = _pad_tc(weights.T)
    w2 = w128.reshape(2 * weights.shape[0], EMB_DIM)
    return _gather(x * 2, w2, cols)


def kernel(x, weights):
    n_rows, cols = x.shape
    assert n_rows % (NUM_WORKERS * N_BUF) == 0
    assert cols <= 128
    return _embed(x.astype(jnp.int32), weights, cols)
